# baseline (speedup 1.0000x reference)
import jax
import jax.numpy as jnp
from jax.experimental import pallas as pl

TOP_K = 8
N_EXPERTS = 64
HIDDEN = 4096

BT = 1024


def _gate_kernel(x_ref, w_ref_in, idx_ref, w_ref):
    logits = jax.lax.dot_general(
        x_ref[...],
        w_ref_in[...],
        (((1,), (1,)), ((), ())),
        preferred_element_type=jnp.float32,
    )
    s = jax.nn.sigmoid(logits).T

    iota = jax.lax.broadcasted_iota(jnp.int32, (N_EXPERTS, BT), 0).astype(
        jnp.float32
    )
    vals = []
    idxs = []
    for _ in range(TOP_K):
        m = jnp.max(s, axis=0, keepdims=True)
        hit = s >= m
        idx = jnp.min(jnp.where(hit, iota, float(N_EXPERTS)), axis=0, keepdims=True)
        vals.append(m)
        idxs.append(idx)
        s = jnp.where(iota == idx, -1.0, s)

    topv = jnp.concatenate(vals, axis=0)
    topi = jnp.concatenate(idxs, axis=0)
    denom = jnp.sum(topv, axis=0, keepdims=True) + 1e-20
    idx_ref[...] = topi.T.astype(jnp.int32)
    w_ref[...] = (topv / denom).T


@jax.jit
def _gate(flat, w):
    n_tokens = flat.shape[0]
    grid = (n_tokens // BT,)
    return pl.pallas_call(
        _gate_kernel,
        grid=grid,
        in_specs=[
            pl.BlockSpec((BT, HIDDEN), lambda i: (i, 0)),
            pl.BlockSpec((N_EXPERTS, HIDDEN), lambda i: (0, 0)),
        ],
        out_specs=[
            pl.BlockSpec((BT, TOP_K), lambda i: (i, 0)),
            pl.BlockSpec((BT, TOP_K), lambda i: (i, 0)),
        ],
        out_shape=[
            jax.ShapeDtypeStruct((n_tokens, TOP_K), jnp.int32),
            jax.ShapeDtypeStruct((n_tokens, TOP_K), jnp.float32),
        ],
    )(flat, w)


def kernel(hidden_states, W):
    bsz, seq_len, h = hidden_states.shape
    flat = hidden_states.reshape(-1, h)
    topk_idx, topk_weight = _gate(flat, W)
    return (topk_idx, topk_weight)
